# baseline (speedup 1.0000x reference)
import functools

import jax
import jax.numpy as jnp
import numpy as np
from jax.experimental import pallas as pl
from jax.experimental.pallas import tpu as pltpu

_N_NODES = 10000
_N_EDGES = 160000
_DIM = 128
_HEADS = 8
_DNP = _DIM // _HEADS
_TEMP = float(np.sqrt(_DNP))
_NP_PAD = 10240

_BE = 2000
_BN = 1024


def _head_max(x):
    m = x
    for k in (8, 16, 32, 64):
        m = jnp.maximum(m, jnp.roll(m, k, axis=-1))
    return m


def _head_sum(x):
    s = x
    for k in (8, 16, 32, 64):
        s = s + jnp.roll(s, k, axis=-1)
    return s


def _edge_kernel(xi_ref, ef_ref, efr_ref, xj_ref, match_ref,
                 w1t_ref, b1_ref, w2t_ref, b2_ref,
                 wqt_ref, bq_ref, wkt_ref, bk_ref, wvt_ref, bv_ref,
                 m1_ref, a1_ref, m2_ref, a2_ref,
                 ue_ref, prob_ref, wgt_ref, *, pre_relu):
    xi = xi_ref[...]
    ef = ef_ref[...]
    efr = efr_ref[...] * match_ref[...]
    xj = xj_ref[...]
    if pre_relu:
        ef = jnp.maximum(ef, 0.0)
        efr = jnp.maximum(efr, 0.0)

    f32 = jnp.float32
    dot = functools.partial(jnp.dot, preferred_element_type=f32)

    w1t = w1t_ref[...]
    h = (dot(xi, w1t[0:128]) + dot(ef, w1t[128:256])
         + dot(efr, w1t[256:384]) + dot(xj, w1t[384:512]) + b1_ref[...])
    h = jnp.maximum(h, 0.0)
    ue = dot(h, w2t_ref[...]) + b2_ref[...]
    ue_ref[...] = ue

    q = dot(xi, wqt_ref[...]) + bq_ref[...]
    kk = dot(ef, wkt_ref[...]) + bk_ref[...]
    v = dot(xj, wvt_ref[...]) + bv_ref[...]

    m1 = m1_ref[...]
    a = dot(q, m1[0:128]) + dot(kk, m1[128:256]) + a1_ref[...]
    a = jnp.maximum(a, 0.0)
    att = dot(a, m2_ref[...]) + a2_ref[...]
    att = att * (1.0 / _TEMP)

    mx = _head_max(att)
    e = jnp.exp(att - mx)
    s = _head_sum(e)
    prob = e / s
    prob_ref[...] = prob
    wgt_ref[...] = prob * v


def _edge_compute(xi, ef, efr, xj, match, wp, pre_relu):
    grid = _N_EDGES // _BE
    bspec_e = pl.BlockSpec((_BE, _DIM), lambda i: (i, 0))
    bspec_m = pl.BlockSpec((_BE, 1), lambda i: (i, 0))

    def wspec(arr):
        return pl.BlockSpec(arr.shape, lambda i: tuple(0 for _ in arr.shape))

    weights = [wp['eu_W1T'], wp['eu_b1'], wp['eu_W2T'], wp['eu_b2'],
               wp['WqT'], wp['bq'], wp['WkT'], wp['bk'], wp['WvT'], wp['bv'],
               wp['M1'], wp['a1'], wp['M2'], wp['a2']]
    out_shape = [jax.ShapeDtypeStruct((_N_EDGES, _DIM), jnp.float32)] * 3
    return pl.pallas_call(
        functools.partial(_edge_kernel, pre_relu=pre_relu),
        grid=grid,
        in_specs=[bspec_e] * 4 + [bspec_m] + [wspec(w) for w in weights],
        out_specs=[bspec_e] * 3,
        out_shape=out_shape,
    )(xi, ef, efr, xj, match, *weights)


def _node_kernel(x_ref, agg_ref, so0_ref, so1_ref, si0_ref, si1_ref,
                 ico_ref, ici_ref, mask_ref,
                 nw1t_ref, nb1_ref, nw2t_ref, nb2_ref, eawt_ref, eab_ref,
                 out_ref):
    f32 = jnp.float32
    dot = functools.partial(jnp.dot, preferred_element_type=f32)
    x = x_ref[...]
    agg = agg_ref[...]
    agg = jnp.where(mask_ref[...] > 0.0, agg, 0.0)
    nw1t = nw1t_ref[...]
    h = dot(x, nw1t[0:128]) + dot(agg, nw1t[128:256]) + nb1_ref[...]
    h = jnp.maximum(h, 0.0)
    un = dot(h, nw2t_ref[...]) + nb2_ref[...]

    mean_out = (so0_ref[...] + so1_ref[...]) * ico_ref[...]
    mean_in = (si0_ref[...] + si1_ref[...]) * ici_ref[...]
    eawt = eawt_ref[...]
    logits = dot(mean_out, eawt[0:128]) + dot(mean_in, eawt[128:256]) + eab_ref[...]
    ea = jax.nn.sigmoid(logits)
    out_ref[...] = jnp.maximum(un, 0.0) * ea


def _node_compute(x_pad, agg, so0, so1, si0, si1, ico, ici, mask, wp):
    grid = _NP_PAD // _BN
    bspec_n = pl.BlockSpec((_BN, _DIM), lambda i: (i, 0))
    bspec_1 = pl.BlockSpec((_BN, 1), lambda i: (i, 0))

    def wspec(arr):
        return pl.BlockSpec(arr.shape, lambda i: tuple(0 for _ in arr.shape))

    weights = [wp['nu_W1T'], wp['nu_b1'], wp['nu_W2T'], wp['nu_b2'],
               wp['ea_WT'], wp['ea_b']]
    return pl.pallas_call(
        _node_kernel,
        grid=grid,
        in_specs=[bspec_n] * 6 + [bspec_1] * 3 + [wspec(w) for w in weights],
        out_specs=bspec_n,
        out_shape=jax.ShapeDtypeStruct((_NP_PAD, _DIM), jnp.float32),
    )(x_pad, agg, so0, so1, si0, si1, ico, ici, mask, *weights)


def _prep_weights(p):
    eye8 = jnp.eye(_HEADS, dtype=jnp.float32)
    return {
        'eu_W1T': p['eu_W1'].T, 'eu_b1': p['eu_b1'][None, :],
        'eu_W2T': p['eu_W2'].T, 'eu_b2': p['eu_b2'][None, :],
        'WqT': p['Wq'].T, 'bq': p['bq'][None, :],
        'WkT': p['Wk'].T, 'bk': p['bk'][None, :],
        'WvT': p['Wv'].T, 'bv': p['bv'][None, :],
        'M1': jnp.kron(p['att_W1'].T, eye8),
        'a1': jnp.repeat(p['att_b1'], _HEADS)[None, :],
        'M2': jnp.kron(p['att_W2'].T, eye8),
        'a2': jnp.repeat(p['att_b2'], _HEADS)[None, :],
        'nu_W1T': p['nu_W1'].T, 'nu_b1': p['nu_b1'][None, :],
        'nu_W2T': p['nu_W2'].T, 'nu_b2': p['nu_b2'][None, :],
        'ea_WT': p['ea_W'].T, 'ea_b': p['ea_b'][None, :],
    }


def kernel(x, edge_feature, edge_index, params):
    row, col = edge_index[0], edge_index[1]
    E = _N_EDGES

    keys = row * _N_NODES + col
    rev = col * _N_NODES + row
    order = jnp.argsort(keys)
    sk = keys[order]
    pos = jnp.clip(jnp.searchsorted(sk, rev), 0, E - 1)
    match = sk[pos] == rev
    rev_idx = jnp.where(match, order[pos], 0)
    match_f = match.astype(jnp.float32)[:, None]

    row_sorted = sk // _N_NODES
    node_ids = jnp.arange(_N_NODES + 1, dtype=jnp.int32)
    bnd_out = jnp.searchsorted(row_sorted, node_ids)
    cnt_out = (bnd_out[1:] - bnd_out[:-1]).astype(jnp.float32)
    order2 = jnp.argsort(col)
    col_sorted = col[order2]
    bnd_in = jnp.searchsorted(col_sorted, node_ids)
    cnt_in = (bnd_in[1:] - bnd_in[:-1]).astype(jnp.float32)

    def pad_n(v):
        return jnp.pad(v, ((0, _NP_PAD - _N_NODES), (0, 0)))

    ico = pad_n((1.0 / jnp.maximum(cnt_out, 1.0))[:, None])
    ici = pad_n((1.0 / jnp.maximum(cnt_in, 1.0))[:, None])
    mask_out = pad_n((cnt_out > 0).astype(jnp.float32)[:, None])

    nf, ef = x, edge_feature
    probs = []
    for li, p in enumerate(params):
        wp = _prep_weights(p)
        pre_relu = li > 0
        ef_for_gather = jnp.maximum(ef, 0.0) if pre_relu else ef
        xi = nf[row]
        xj = nf[col]
        efr = ef_for_gather[rev_idx]
        ef_in = ef_for_gather

        ue, prob, wgt = _edge_compute(xi, ef_in, efr, xj, match_f, wp,
                                      pre_relu=False)
        probs.append(prob.reshape(E, _DNP, _HEADS))

        agg = jax.ops.segment_max(wgt, row, num_segments=_N_NODES)
        agg = jnp.where(jnp.isfinite(agg), agg, 0.0)
        sum_out = jax.ops.segment_sum(ue, row, num_segments=_N_NODES)
        sum_in = jax.ops.segment_sum(ue, col, num_segments=_N_NODES)
        zeros = jnp.zeros((_NP_PAD, _DIM), jnp.float32)

        fn = _node_compute(pad_n(nf), pad_n(agg), pad_n(sum_out), zeros,
                           pad_n(sum_in), zeros, ico, ici, mask_out, wp)
        nf = fn[:_N_NODES]
        ef = ue
    return (nf, ef, probs)
